# baseline (speedup 1.0000x reference)
import functools

import jax
import jax.numpy as jnp
from jax import lax
from jax.experimental import pallas as pl
from jax.experimental.pallas import tpu as pltpu
from jax.experimental.pallas import tpu_sc as plsc

N = 2048
K = 8
OUT_PER = 8
BR = 256
NC = 2
NS = 16
NW = NC * NS
TC_ROWS = 256
SC_ROWS = N - TC_ROWS
SC_RPW = SC_ROWS // NW
CHUNKS = N // 16
QI = 4
INF = float("inf")


def _sc_body(x1h, y1h, x2h, y2h, wth, bth, outh,
             x1v, y1v, x2v, y2v, vxv, vyv, wtv, btv, fbuf, outv):
    wid = lax.axis_index("s") * NC + lax.axis_index("c")
    base_row = TC_ROWS + wid * SC_RPW

    pltpu.sync_copy(x1h, x1v)
    pltpu.sync_copy(y1h, y1v)
    pltpu.sync_copy(x2h, x2v)
    pltpu.sync_copy(y2h, y2v)
    pltpu.sync_copy(wth, wtv)
    pltpu.sync_copy(bth, btv)

    io = lax.iota(jnp.int32, 16)
    khalf = lax.shift_right_logical(io, 3)

    def _vel(c, carry):
        s = pl.ds(c * 16, 16)
        vxv[s] = x2v[s] - x1v[s]
        vyv[s] = y2v[s] - y1v[s]
        return carry
    lax.fori_loop(0, CHUNKS, _vel, 0)

    btile = btv[...]
    w0 = wtv[0, :]
    w1 = wtv[1, :]
    w2 = wtv[2, :]
    w3 = wtv[3, :]

    def _post(a, xi, yi, vxi, vyi, bv):
        gx = plsc.load_gather(x2v, [bv])
        gy = plsc.load_gather(y2v, [bv])
        gvx = plsc.load_gather(vxv, [bv])
        gvy = plsc.load_gather(vyv, [bv])
        fbuf[pl.ds(0, 16)] = gx - xi
        fbuf[pl.ds(16, 16)] = gy - yi
        fbuf[pl.ds(32, 16)] = gvx - vxi
        fbuf[pl.ds(48, 16)] = gvy - vyi

        for g in range(4):
            sel = khalf + (2 * g)
            acc = btile
            acc = acc + plsc.load_gather(fbuf, [sel]) * w0
            acc = acc + plsc.load_gather(fbuf, [sel + 16]) * w1
            acc = acc + plsc.load_gather(fbuf, [sel + 32]) * w2
            acc = acc + plsc.load_gather(fbuf, [sel + 48]) * w3
            outv[a, pl.ds(g * 16, 16)] = jnp.maximum(acc, 0.0)

    def _agent_group(p, carry):
        ags = [p * QI + q for q in range(QI)]
        ivs = [jnp.full((16,), base_row + a, jnp.int32) for a in ags]
        xis = [plsc.load_gather(x2v, [iv]) for iv in ivs]
        yis = [plsc.load_gather(y2v, [iv]) for iv in ivs]
        vxis = [plsc.load_gather(vxv, [iv]) for iv in ivs]
        vyis = [plsc.load_gather(vyv, [iv]) for iv in ivs]

        def _chunk(c, bkv):
            s = pl.ds(c * 16, 16)
            civ = io + c * 16
            xs = x2v[s]
            ys = y2v[s]
            nxt = []
            for q in range(QI):
                bk, bvv = bkv[2 * q], bkv[2 * q + 1]
                dx = xs - xis[q]
                dy = ys - yis[q]
                d = dx * dx + dy * dy
                d = jnp.where(civ == ivs[q], INF, d)
                ck, cw = plsc.sort_key_val(d, civ)
                rk = lax.rev(ck, (0,))
                rv = lax.rev(cw, (0,))
                m = bk <= rk
                nk, nv = plsc.sort_key_val(
                    jnp.where(m, bk, rk), jnp.where(m, bvv, rv))
                nxt += [nk, nv]
            return tuple(nxt)

        inf16 = jnp.full((16,), INF, jnp.float32)
        z16 = jnp.zeros((16,), jnp.int32)
        res = lax.fori_loop(0, CHUNKS, _chunk, (inf16, z16) * QI)
        for q in range(QI):
            _post(ags[q], xis[q], yis[q], vxis[q], vyis[q], res[2 * q + 1])
        return carry

    lax.fori_loop(0, SC_RPW // QI, _agent_group, 0)

    pltpu.sync_copy(outv, outh.at[pl.ds(wid * SC_RPW, SC_RPW)])


def _sc_run(x1, y1, x2, y2, W, b):
    wt = jnp.tile(W.T, (1, 2))
    bt = jnp.tile(b, 2)
    mesh = plsc.VectorSubcoreMesh(
        core_axis_name="c", subcore_axis_name="s",
        num_cores=NC, num_subcores=NS)
    kern = functools.partial(
        pl.kernel,
        out_type=jax.ShapeDtypeStruct((SC_ROWS, K * OUT_PER), jnp.float32),
        mesh=mesh,
        compiler_params=pltpu.CompilerParams(
            use_tc_tiling_on_sc=False, needs_layout_passes=False),
        scratch_types=[
            pltpu.VMEM((N,), jnp.float32),
            pltpu.VMEM((N,), jnp.float32),
            pltpu.VMEM((N,), jnp.float32),
            pltpu.VMEM((N,), jnp.float32),
            pltpu.VMEM((N,), jnp.float32),
            pltpu.VMEM((N,), jnp.float32),
            pltpu.VMEM((4, 16), jnp.float32),
            pltpu.VMEM((16,), jnp.float32),
            pltpu.VMEM((64,), jnp.float32),
            pltpu.VMEM((SC_RPW, K * OUT_PER), jnp.float32),
        ],
    )(_sc_body)
    return kern(x1, y1, x2, y2, wt, bt)


def _tc_body(x1r, y1r, x2c, y2c, x2r, y2r, wt, b2, out_ref):
    i = pl.program_id(0)
    base = i * BR

    col = lax.broadcasted_iota(jnp.int32, (BR, N), 1)
    row = base + lax.broadcasted_iota(jnp.int32, (BR, N), 0)

    relx = x2r[...] - x2c[...]
    rely = y2r[...] - y2c[...]
    dist = jnp.sqrt(relx * relx + rely * rely)
    dist = jnp.where(col == row, jnp.inf, dist)

    vxr = x2r[...] - x1r[...]
    vyr = y2r[...] - y1r[...]
    ptab = jnp.concatenate([x2r[...], y2r[...], vxr, vyr], axis=0).T

    rowhot = (col == row).astype(jnp.float32)
    self4 = jnp.dot(rowhot, ptab, preferred_element_type=jnp.float32)

    for k in range(K):
        m = jnp.min(dist, axis=1, keepdims=True)
        cand = jnp.where(dist == m, col, N)
        idx = jnp.min(cand, axis=1, keepdims=True)
        onehot = (col == idx).astype(jnp.float32)
        feats = jnp.dot(onehot, ptab, preferred_element_type=jnp.float32)
        rel = feats - self4
        emb = jnp.maximum(
            jnp.dot(rel, wt[...], preferred_element_type=jnp.float32)
            + b2[...], 0.0)
        out_ref[:, k * OUT_PER:(k + 1) * OUT_PER] = emb
        if k != K - 1:
            dist = jnp.where(col == idx, jnp.inf, dist)


def _tc_run(x1, y1, x2, y2, W, b):
    x1r = x1.reshape(1, N)
    y1r = y1.reshape(1, N)
    x2r = x2.reshape(1, N)
    y2r = y2.reshape(1, N)
    x2c = x2.reshape(N, 1)
    y2c = y2.reshape(N, 1)
    wt = W.T
    b2 = b.reshape(1, OUT_PER)

    grid = (TC_ROWS // BR,)
    full_row = pl.BlockSpec((1, N), lambda i: (0, 0))
    col_blk = pl.BlockSpec((BR, 1), lambda i: (i, 0))
    return pl.pallas_call(
        _tc_body,
        grid=grid,
        in_specs=[
            full_row, full_row,
            col_blk, col_blk,
            full_row, full_row,
            pl.BlockSpec((4, OUT_PER), lambda i: (0, 0)),
            pl.BlockSpec((1, OUT_PER), lambda i: (0, 0)),
        ],
        out_specs=pl.BlockSpec((BR, K * OUT_PER), lambda i: (i, 0)),
        out_shape=jax.ShapeDtypeStruct((TC_ROWS, K * OUT_PER), jnp.float32),
    )(x1r, y1r, x2c, y2c, x2r, y2r, wt, b2)


@jax.jit
def _run(obs1, obs2, W, b):
    x1 = obs1[:, 0]
    y1 = obs1[:, 1]
    x2 = obs2[:, 0]
    y2 = obs2[:, 1]
    sc_out = _sc_run(x1, y1, x2, y2, W, b)
    tc_out = _tc_run(x1, y1, x2, y2, W, b)
    return jnp.concatenate([tc_out, sc_out], axis=0)


def kernel(_, obs1, obs2, W, b):
    return _run(obs1, obs2, W, b)


# speedup vs baseline: 1.0557x; 1.0557x over previous
import functools

import jax
import jax.numpy as jnp
from jax import lax
from jax.experimental import pallas as pl
from jax.experimental.pallas import tpu as pltpu
from jax.experimental.pallas import tpu_sc as plsc

N = 2048
K = 8
OUT_PER = 8
BR = 256
NC = 2
NS = 16
NW = NC * NS
TC_ROWS = 512
SC_ROWS = N - TC_ROWS
SC_RPW = SC_ROWS // NW
CHUNKS = N // 16
QI = 8
INF = float("inf")


def _sc_body(x1h, y1h, x2h, y2h, wth, bth, outh,
             x1v, y1v, x2v, y2v, vxv, vyv, wtv, btv, fbuf, outv):
    wid = lax.axis_index("s") * NC + lax.axis_index("c")
    base_row = TC_ROWS + wid * SC_RPW

    pltpu.sync_copy(x1h, x1v)
    pltpu.sync_copy(y1h, y1v)
    pltpu.sync_copy(x2h, x2v)
    pltpu.sync_copy(y2h, y2v)
    pltpu.sync_copy(wth, wtv)
    pltpu.sync_copy(bth, btv)

    io = lax.iota(jnp.int32, 16)
    khalf = lax.shift_right_logical(io, 3)

    def _vel(c, carry):
        s = pl.ds(c * 16, 16)
        vxv[s] = x2v[s] - x1v[s]
        vyv[s] = y2v[s] - y1v[s]
        return carry
    lax.fori_loop(0, CHUNKS, _vel, 0)

    btile = btv[...]
    w0 = wtv[0, :]
    w1 = wtv[1, :]
    w2 = wtv[2, :]
    w3 = wtv[3, :]

    def _post(a, xi, yi, vxi, vyi, bv):
        gx = plsc.load_gather(x2v, [bv])
        gy = plsc.load_gather(y2v, [bv])
        gvx = plsc.load_gather(vxv, [bv])
        gvy = plsc.load_gather(vyv, [bv])
        fbuf[pl.ds(0, 16)] = gx - xi
        fbuf[pl.ds(16, 16)] = gy - yi
        fbuf[pl.ds(32, 16)] = gvx - vxi
        fbuf[pl.ds(48, 16)] = gvy - vyi

        for g in range(4):
            sel = khalf + (2 * g)
            acc = btile
            acc = acc + plsc.load_gather(fbuf, [sel]) * w0
            acc = acc + plsc.load_gather(fbuf, [sel + 16]) * w1
            acc = acc + plsc.load_gather(fbuf, [sel + 32]) * w2
            acc = acc + plsc.load_gather(fbuf, [sel + 48]) * w3
            outv[a, pl.ds(g * 16, 16)] = jnp.maximum(acc, 0.0)

    def _agent_group(p, carry):
        ags = [p * QI + q for q in range(QI)]
        ivs = [jnp.full((16,), base_row + a, jnp.int32) for a in ags]
        xis = [plsc.load_gather(x2v, [iv]) for iv in ivs]
        yis = [plsc.load_gather(y2v, [iv]) for iv in ivs]
        vxis = [plsc.load_gather(vxv, [iv]) for iv in ivs]
        vyis = [plsc.load_gather(vyv, [iv]) for iv in ivs]

        def _chunk(c, bkv):
            s = pl.ds(c * 16, 16)
            civ = io + c * 16
            xs = x2v[s]
            ys = y2v[s]
            nxt = []
            for q in range(QI):
                bk, bvv = bkv[2 * q], bkv[2 * q + 1]
                dx = xs - xis[q]
                dy = ys - yis[q]
                d = dx * dx + dy * dy
                d = jnp.where(civ == ivs[q], INF, d)
                ck, cw = plsc.sort_key_val(d, civ)
                rk = lax.rev(ck, (0,))
                rv = lax.rev(cw, (0,))
                m = bk <= rk
                nk, nv = plsc.sort_key_val(
                    jnp.where(m, bk, rk), jnp.where(m, bvv, rv))
                nxt += [nk, nv]
            return tuple(nxt)

        inf16 = jnp.full((16,), INF, jnp.float32)
        z16 = jnp.zeros((16,), jnp.int32)
        res = lax.fori_loop(0, CHUNKS, _chunk, (inf16, z16) * QI)
        for q in range(QI):
            _post(ags[q], xis[q], yis[q], vxis[q], vyis[q], res[2 * q + 1])
        return carry

    lax.fori_loop(0, SC_RPW // QI, _agent_group, 0)

    pltpu.sync_copy(outv, outh.at[pl.ds(wid * SC_RPW, SC_RPW)])


def _sc_run(x1, y1, x2, y2, W, b):
    wt = jnp.tile(W.T, (1, 2))
    bt = jnp.tile(b, 2)
    mesh = plsc.VectorSubcoreMesh(
        core_axis_name="c", subcore_axis_name="s",
        num_cores=NC, num_subcores=NS)
    kern = functools.partial(
        pl.kernel,
        out_type=jax.ShapeDtypeStruct((SC_ROWS, K * OUT_PER), jnp.float32),
        mesh=mesh,
        compiler_params=pltpu.CompilerParams(
            use_tc_tiling_on_sc=False, needs_layout_passes=False),
        scratch_types=[
            pltpu.VMEM((N,), jnp.float32),
            pltpu.VMEM((N,), jnp.float32),
            pltpu.VMEM((N,), jnp.float32),
            pltpu.VMEM((N,), jnp.float32),
            pltpu.VMEM((N,), jnp.float32),
            pltpu.VMEM((N,), jnp.float32),
            pltpu.VMEM((4, 16), jnp.float32),
            pltpu.VMEM((16,), jnp.float32),
            pltpu.VMEM((64,), jnp.float32),
            pltpu.VMEM((SC_RPW, K * OUT_PER), jnp.float32),
        ],
    )(_sc_body)
    return kern(x1, y1, x2, y2, wt, bt)


def _tc_body(x1r, y1r, x2c, y2c, x2r, y2r, wt, b2, out_ref):
    i = pl.program_id(0)
    base = i * BR

    col = lax.broadcasted_iota(jnp.int32, (BR, N), 1)
    row = base + lax.broadcasted_iota(jnp.int32, (BR, N), 0)

    relx = x2r[...] - x2c[...]
    rely = y2r[...] - y2c[...]
    dist = jnp.sqrt(relx * relx + rely * rely)
    dist = jnp.where(col == row, jnp.inf, dist)

    vxr = x2r[...] - x1r[...]
    vyr = y2r[...] - y1r[...]
    ptab = jnp.concatenate([x2r[...], y2r[...], vxr, vyr], axis=0).T

    rowhot = (col == row).astype(jnp.float32)
    self4 = jnp.dot(rowhot, ptab, preferred_element_type=jnp.float32)

    for k in range(K):
        m = jnp.min(dist, axis=1, keepdims=True)
        cand = jnp.where(dist == m, col, N)
        idx = jnp.min(cand, axis=1, keepdims=True)
        onehot = (col == idx).astype(jnp.float32)
        feats = jnp.dot(onehot, ptab, preferred_element_type=jnp.float32)
        rel = feats - self4
        emb = jnp.maximum(
            jnp.dot(rel, wt[...], preferred_element_type=jnp.float32)
            + b2[...], 0.0)
        out_ref[:, k * OUT_PER:(k + 1) * OUT_PER] = emb
        if k != K - 1:
            dist = jnp.where(col == idx, jnp.inf, dist)


def _tc_run(x1, y1, x2, y2, W, b):
    x1r = x1.reshape(1, N)
    y1r = y1.reshape(1, N)
    x2r = x2.reshape(1, N)
    y2r = y2.reshape(1, N)
    x2c = x2.reshape(N, 1)
    y2c = y2.reshape(N, 1)
    wt = W.T
    b2 = b.reshape(1, OUT_PER)

    grid = (TC_ROWS // BR,)
    full_row = pl.BlockSpec((1, N), lambda i: (0, 0))
    col_blk = pl.BlockSpec((BR, 1), lambda i: (i, 0))
    return pl.pallas_call(
        _tc_body,
        grid=grid,
        in_specs=[
            full_row, full_row,
            col_blk, col_blk,
            full_row, full_row,
            pl.BlockSpec((4, OUT_PER), lambda i: (0, 0)),
            pl.BlockSpec((1, OUT_PER), lambda i: (0, 0)),
        ],
        out_specs=pl.BlockSpec((BR, K * OUT_PER), lambda i: (i, 0)),
        out_shape=jax.ShapeDtypeStruct((TC_ROWS, K * OUT_PER), jnp.float32),
    )(x1r, y1r, x2c, y2c, x2r, y2r, wt, b2)


@jax.jit
def _run(obs1, obs2, W, b):
    x1 = obs1[:, 0]
    y1 = obs1[:, 1]
    x2 = obs2[:, 0]
    y2 = obs2[:, 1]
    sc_out = _sc_run(x1, y1, x2, y2, W, b)
    tc_out = _tc_run(x1, y1, x2, y2, W, b)
    return jnp.concatenate([tc_out, sc_out], axis=0)


def kernel(_, obs1, obs2, W, b):
    return _run(obs1, obs2, W, b)


# speedup vs baseline: 1.0650x; 1.0089x over previous
import functools

import jax
import jax.numpy as jnp
from jax import lax
from jax.experimental import pallas as pl
from jax.experimental.pallas import tpu as pltpu
from jax.experimental.pallas import tpu_sc as plsc

N = 2048
K = 8
OUT_PER = 8
BR = 256
NC = 2
NS = 16
NW = NC * NS
TC_ROWS = 512
SC_ROWS = N - TC_ROWS
SC_RPW = SC_ROWS // NW
CHUNKS = N // 16
QI = 4
INF = float("inf")


def _sc_body(x1h, y1h, x2h, y2h, wth, bth, outh,
             x1v, y1v, x2v, y2v, vxv, vyv, wtv, btv, fbuf, outv):
    wid = lax.axis_index("s") * NC + lax.axis_index("c")
    base_row = TC_ROWS + wid * SC_RPW

    pltpu.sync_copy(x1h, x1v)
    pltpu.sync_copy(y1h, y1v)
    pltpu.sync_copy(x2h, x2v)
    pltpu.sync_copy(y2h, y2v)
    pltpu.sync_copy(wth, wtv)
    pltpu.sync_copy(bth, btv)

    io = lax.iota(jnp.int32, 16)
    khalf = lax.shift_right_logical(io, 3)

    def _vel(c, carry):
        s = pl.ds(c * 16, 16)
        vxv[s] = x2v[s] - x1v[s]
        vyv[s] = y2v[s] - y1v[s]
        return carry
    lax.fori_loop(0, CHUNKS, _vel, 0)

    btile = btv[...]
    w0 = wtv[0, :]
    w1 = wtv[1, :]
    w2 = wtv[2, :]
    w3 = wtv[3, :]

    def _post(a, xi, yi, vxi, vyi, bv):
        gx = plsc.load_gather(x2v, [bv])
        gy = plsc.load_gather(y2v, [bv])
        gvx = plsc.load_gather(vxv, [bv])
        gvy = plsc.load_gather(vyv, [bv])
        fbuf[pl.ds(0, 16)] = gx - xi
        fbuf[pl.ds(16, 16)] = gy - yi
        fbuf[pl.ds(32, 16)] = gvx - vxi
        fbuf[pl.ds(48, 16)] = gvy - vyi

        for g in range(4):
            sel = khalf + (2 * g)
            acc = btile
            acc = acc + plsc.load_gather(fbuf, [sel]) * w0
            acc = acc + plsc.load_gather(fbuf, [sel + 16]) * w1
            acc = acc + plsc.load_gather(fbuf, [sel + 32]) * w2
            acc = acc + plsc.load_gather(fbuf, [sel + 48]) * w3
            outv[a, pl.ds(g * 16, 16)] = jnp.maximum(acc, 0.0)

    def _agent_group(p, carry):
        ags = [p * QI + q for q in range(QI)]
        ivs = [jnp.full((16,), base_row + a, jnp.int32) for a in ags]
        xis = [plsc.load_gather(x2v, [iv]) for iv in ivs]
        yis = [plsc.load_gather(y2v, [iv]) for iv in ivs]
        vxis = [plsc.load_gather(vxv, [iv]) for iv in ivs]
        vyis = [plsc.load_gather(vyv, [iv]) for iv in ivs]

        def _chunk(c, bkv):
            s = pl.ds(c * 16, 16)
            civ = io + c * 16
            xs = x2v[s]
            ys = y2v[s]
            nxt = []
            for q in range(QI):
                bk, bvv = bkv[2 * q], bkv[2 * q + 1]
                dx = xs - xis[q]
                dy = ys - yis[q]
                d = dx * dx + dy * dy
                d = jnp.where(civ == ivs[q], INF, d)
                ck, cw = plsc.sort_key_val(d, civ)
                rk = lax.rev(ck, (0,))
                rv = lax.rev(cw, (0,))
                m = bk <= rk
                nk, nv = plsc.sort_key_val(
                    jnp.where(m, bk, rk), jnp.where(m, bvv, rv))
                nxt += [nk, nv]
            return tuple(nxt)

        inf16 = jnp.full((16,), INF, jnp.float32)
        z16 = jnp.zeros((16,), jnp.int32)
        res = lax.fori_loop(0, CHUNKS, _chunk, (inf16, z16) * QI)
        for q in range(QI):
            _post(ags[q], xis[q], yis[q], vxis[q], vyis[q], res[2 * q + 1])
        return carry

    lax.fori_loop(0, SC_RPW // QI, _agent_group, 0)

    pltpu.sync_copy(outv, outh.at[pl.ds(wid * SC_RPW, SC_RPW)])


def _sc_run(x1, y1, x2, y2, W, b):
    wt = jnp.tile(W.T, (1, 2))
    bt = jnp.tile(b, 2)
    mesh = plsc.VectorSubcoreMesh(
        core_axis_name="c", subcore_axis_name="s",
        num_cores=NC, num_subcores=NS)
    kern = functools.partial(
        pl.kernel,
        out_type=jax.ShapeDtypeStruct((SC_ROWS, K * OUT_PER), jnp.float32),
        mesh=mesh,
        compiler_params=pltpu.CompilerParams(
            use_tc_tiling_on_sc=False, needs_layout_passes=False),
        scratch_types=[
            pltpu.VMEM((N,), jnp.float32),
            pltpu.VMEM((N,), jnp.float32),
            pltpu.VMEM((N,), jnp.float32),
            pltpu.VMEM((N,), jnp.float32),
            pltpu.VMEM((N,), jnp.float32),
            pltpu.VMEM((N,), jnp.float32),
            pltpu.VMEM((4, 16), jnp.float32),
            pltpu.VMEM((16,), jnp.float32),
            pltpu.VMEM((64,), jnp.float32),
            pltpu.VMEM((SC_RPW, K * OUT_PER), jnp.float32),
        ],
    )(_sc_body)
    return kern(x1, y1, x2, y2, wt, bt)


def _tc_body(x1r, y1r, x2c, y2c, x2r, y2r, wt, b2, out_ref):
    i = pl.program_id(0)
    base = i * BR

    col = lax.broadcasted_iota(jnp.int32, (BR, N), 1)
    row = base + lax.broadcasted_iota(jnp.int32, (BR, N), 0)

    relx = x2r[...] - x2c[...]
    rely = y2r[...] - y2c[...]
    dist = jnp.sqrt(relx * relx + rely * rely)
    dist = jnp.where(col == row, jnp.inf, dist)

    vxr = x2r[...] - x1r[...]
    vyr = y2r[...] - y1r[...]
    ptab = jnp.concatenate([x2r[...], y2r[...], vxr, vyr], axis=0).T

    rowhot = (col == row).astype(jnp.float32)
    self4 = jnp.dot(rowhot, ptab, preferred_element_type=jnp.float32)

    for k in range(K):
        idx = jnp.argmin(dist, axis=1).astype(jnp.int32).reshape(BR, 1)
        onehot = (col == idx).astype(jnp.float32)
        feats = jnp.dot(onehot, ptab, preferred_element_type=jnp.float32)
        rel = feats - self4
        emb = jnp.maximum(
            jnp.dot(rel, wt[...], preferred_element_type=jnp.float32)
            + b2[...], 0.0)
        out_ref[:, k * OUT_PER:(k + 1) * OUT_PER] = emb
        if k != K - 1:
            dist = jnp.where(col == idx, jnp.inf, dist)


def _tc_run(x1, y1, x2, y2, W, b):
    x1r = x1.reshape(1, N)
    y1r = y1.reshape(1, N)
    x2r = x2.reshape(1, N)
    y2r = y2.reshape(1, N)
    x2c = x2.reshape(N, 1)
    y2c = y2.reshape(N, 1)
    wt = W.T
    b2 = b.reshape(1, OUT_PER)

    grid = (TC_ROWS // BR,)
    full_row = pl.BlockSpec((1, N), lambda i: (0, 0))
    col_blk = pl.BlockSpec((BR, 1), lambda i: (i, 0))
    return pl.pallas_call(
        _tc_body,
        grid=grid,
        in_specs=[
            full_row, full_row,
            col_blk, col_blk,
            full_row, full_row,
            pl.BlockSpec((4, OUT_PER), lambda i: (0, 0)),
            pl.BlockSpec((1, OUT_PER), lambda i: (0, 0)),
        ],
        out_specs=pl.BlockSpec((BR, K * OUT_PER), lambda i: (i, 0)),
        out_shape=jax.ShapeDtypeStruct((TC_ROWS, K * OUT_PER), jnp.float32),
    )(x1r, y1r, x2c, y2c, x2r, y2r, wt, b2)


@jax.jit
def _run(obs1, obs2, W, b):
    x1 = obs1[:, 0]
    y1 = obs1[:, 1]
    x2 = obs2[:, 0]
    y2 = obs2[:, 1]
    sc_out = _sc_run(x1, y1, x2, y2, W, b)
    tc_out = _tc_run(x1, y1, x2, y2, W, b)
    return jnp.concatenate([tc_out, sc_out], axis=0)


def kernel(_, obs1, obs2, W, b):
    return _run(obs1, obs2, W, b)


# speedup vs baseline: 1.1018x; 1.0345x over previous
import functools

import jax
import jax.numpy as jnp
from jax import lax
from jax.experimental import pallas as pl
from jax.experimental.pallas import tpu as pltpu
from jax.experimental.pallas import tpu_sc as plsc

N = 2048
K = 8
OUT_PER = 8
BR = 128
NC = 2
NS = 16
NW = NC * NS
TC_ROWS = 640
SC_ROWS = N - TC_ROWS
SC_RPW = SC_ROWS // NW
CHUNKS = N // 16
QI = 4
INF = float("inf")


def _sc_body(x1h, y1h, x2h, y2h, wth, bth, outh,
             x1v, y1v, x2v, y2v, vxv, vyv, wtv, btv, fbuf, outv):
    wid = lax.axis_index("s") * NC + lax.axis_index("c")
    base_row = TC_ROWS + wid * SC_RPW

    pltpu.sync_copy(x1h, x1v)
    pltpu.sync_copy(y1h, y1v)
    pltpu.sync_copy(x2h, x2v)
    pltpu.sync_copy(y2h, y2v)
    pltpu.sync_copy(wth, wtv)
    pltpu.sync_copy(bth, btv)

    io = lax.iota(jnp.int32, 16)
    khalf = lax.shift_right_logical(io, 3)

    def _vel(c, carry):
        s = pl.ds(c * 16, 16)
        vxv[s] = x2v[s] - x1v[s]
        vyv[s] = y2v[s] - y1v[s]
        return carry
    lax.fori_loop(0, CHUNKS, _vel, 0)

    btile = btv[...]
    w0 = wtv[0, :]
    w1 = wtv[1, :]
    w2 = wtv[2, :]
    w3 = wtv[3, :]

    def _post(a, xi, yi, vxi, vyi, bv):
        gx = plsc.load_gather(x2v, [bv])
        gy = plsc.load_gather(y2v, [bv])
        gvx = plsc.load_gather(vxv, [bv])
        gvy = plsc.load_gather(vyv, [bv])
        fbuf[pl.ds(0, 16)] = gx - xi
        fbuf[pl.ds(16, 16)] = gy - yi
        fbuf[pl.ds(32, 16)] = gvx - vxi
        fbuf[pl.ds(48, 16)] = gvy - vyi

        for g in range(4):
            sel = khalf + (2 * g)
            acc = btile
            acc = acc + plsc.load_gather(fbuf, [sel]) * w0
            acc = acc + plsc.load_gather(fbuf, [sel + 16]) * w1
            acc = acc + plsc.load_gather(fbuf, [sel + 32]) * w2
            acc = acc + plsc.load_gather(fbuf, [sel + 48]) * w3
            outv[a, pl.ds(g * 16, 16)] = jnp.maximum(acc, 0.0)

    def _agent_group(p, carry):
        ags = [p * QI + q for q in range(QI)]
        ivs = [jnp.full((16,), base_row + a, jnp.int32) for a in ags]
        xis = [plsc.load_gather(x2v, [iv]) for iv in ivs]
        yis = [plsc.load_gather(y2v, [iv]) for iv in ivs]
        vxis = [plsc.load_gather(vxv, [iv]) for iv in ivs]
        vyis = [plsc.load_gather(vyv, [iv]) for iv in ivs]

        def _chunk(c, bkv):
            s = pl.ds(c * 16, 16)
            civ = io + c * 16
            xs = x2v[s]
            ys = y2v[s]
            nxt = []
            for q in range(QI):
                bk, bvv = bkv[2 * q], bkv[2 * q + 1]
                dx = xs - xis[q]
                dy = ys - yis[q]
                d = dx * dx + dy * dy
                d = jnp.where(civ == ivs[q], INF, d)
                ck, cw = plsc.sort_key_val(d, civ)
                rk = lax.rev(ck, (0,))
                rv = lax.rev(cw, (0,))
                m = bk <= rk
                nk, nv = plsc.sort_key_val(
                    jnp.where(m, bk, rk), jnp.where(m, bvv, rv))
                nxt += [nk, nv]
            return tuple(nxt)

        inf16 = jnp.full((16,), INF, jnp.float32)
        z16 = jnp.zeros((16,), jnp.int32)
        res = lax.fori_loop(0, CHUNKS, _chunk, (inf16, z16) * QI)
        for q in range(QI):
            _post(ags[q], xis[q], yis[q], vxis[q], vyis[q], res[2 * q + 1])
        return carry

    lax.fori_loop(0, SC_RPW // QI, _agent_group, 0)

    pltpu.sync_copy(outv, outh.at[pl.ds(wid * SC_RPW, SC_RPW)])


def _sc_run(x1, y1, x2, y2, W, b):
    wt = jnp.tile(W.T, (1, 2))
    bt = jnp.tile(b, 2)
    mesh = plsc.VectorSubcoreMesh(
        core_axis_name="c", subcore_axis_name="s",
        num_cores=NC, num_subcores=NS)
    kern = functools.partial(
        pl.kernel,
        out_type=jax.ShapeDtypeStruct((SC_ROWS, K * OUT_PER), jnp.float32),
        mesh=mesh,
        compiler_params=pltpu.CompilerParams(
            use_tc_tiling_on_sc=False, needs_layout_passes=False),
        scratch_types=[
            pltpu.VMEM((N,), jnp.float32),
            pltpu.VMEM((N,), jnp.float32),
            pltpu.VMEM((N,), jnp.float32),
            pltpu.VMEM((N,), jnp.float32),
            pltpu.VMEM((N,), jnp.float32),
            pltpu.VMEM((N,), jnp.float32),
            pltpu.VMEM((4, 16), jnp.float32),
            pltpu.VMEM((16,), jnp.float32),
            pltpu.VMEM((64,), jnp.float32),
            pltpu.VMEM((SC_RPW, K * OUT_PER), jnp.float32),
        ],
    )(_sc_body)
    return kern(x1, y1, x2, y2, wt, bt)


def _tc_body(x1r, y1r, x2c, y2c, x2r, y2r, wt, b2, out_ref):
    i = pl.program_id(0)
    base = i * BR

    col = lax.broadcasted_iota(jnp.int32, (BR, N), 1)
    row = base + lax.broadcasted_iota(jnp.int32, (BR, N), 0)

    relx = x2r[...] - x2c[...]
    rely = y2r[...] - y2c[...]
    dist = jnp.sqrt(relx * relx + rely * rely)
    dist = jnp.where(col == row, jnp.inf, dist)

    vxr = x2r[...] - x1r[...]
    vyr = y2r[...] - y1r[...]
    ptab = jnp.concatenate([x2r[...], y2r[...], vxr, vyr], axis=0).T

    rowhot = (col == row).astype(jnp.float32)
    self4 = jnp.dot(rowhot, ptab, preferred_element_type=jnp.float32)

    for k in range(K):
        idx = jnp.argmin(dist, axis=1).astype(jnp.int32).reshape(BR, 1)
        onehot = (col == idx).astype(jnp.float32)
        feats = jnp.dot(onehot, ptab, preferred_element_type=jnp.float32)
        rel = feats - self4
        emb = jnp.maximum(
            jnp.dot(rel, wt[...], preferred_element_type=jnp.float32)
            + b2[...], 0.0)
        out_ref[:, k * OUT_PER:(k + 1) * OUT_PER] = emb
        if k != K - 1:
            dist = jnp.where(col == idx, jnp.inf, dist)


def _tc_run(x1, y1, x2, y2, W, b):
    x1r = x1.reshape(1, N)
    y1r = y1.reshape(1, N)
    x2r = x2.reshape(1, N)
    y2r = y2.reshape(1, N)
    x2c = x2.reshape(N, 1)
    y2c = y2.reshape(N, 1)
    wt = W.T
    b2 = b.reshape(1, OUT_PER)

    grid = (TC_ROWS // BR,)
    full_row = pl.BlockSpec((1, N), lambda i: (0, 0))
    col_blk = pl.BlockSpec((BR, 1), lambda i: (i, 0))
    return pl.pallas_call(
        _tc_body,
        grid=grid,
        in_specs=[
            full_row, full_row,
            col_blk, col_blk,
            full_row, full_row,
            pl.BlockSpec((4, OUT_PER), lambda i: (0, 0)),
            pl.BlockSpec((1, OUT_PER), lambda i: (0, 0)),
        ],
        out_specs=pl.BlockSpec((BR, K * OUT_PER), lambda i: (i, 0)),
        out_shape=jax.ShapeDtypeStruct((TC_ROWS, K * OUT_PER), jnp.float32),
    )(x1r, y1r, x2c, y2c, x2r, y2r, wt, b2)


@jax.jit
def _run(obs1, obs2, W, b):
    x1 = obs1[:, 0]
    y1 = obs1[:, 1]
    x2 = obs2[:, 0]
    y2 = obs2[:, 1]
    sc_out = _sc_run(x1, y1, x2, y2, W, b)
    tc_out = _tc_run(x1, y1, x2, y2, W, b)
    return jnp.concatenate([tc_out, sc_out], axis=0)


def kernel(_, obs1, obs2, W, b):
    return _run(obs1, obs2, W, b)
